# baseline (speedup 1.0000x reference)
import functools

import jax
import jax.numpy as jnp
from jax import lax
from jax.experimental import pallas as pl
from jax.experimental.pallas import tpu as pltpu
from jax.experimental.pallas import tpu_sc as plsc

N_NUM = 13
N_CAT = 26
CARD = 100000
D = 64
BATCH = 4096
N_TOK = N_NUM + N_CAT

NC = 2
NS = 16
NW = NC * NS
B_PER_W = BATCH // NW
NUM_ROWS_W = B_PER_W * N_NUM
CHUNK = 128
NUM_CHUNKS = NUM_ROWS_W // CHUNK
L = 16


def _num_tokens_tc(x_num, num_weight, num_bias, ln_gamma, ln_beta):
    def body(x_ref, w_ref, b_ref, g_ref, be_ref, o_ref):
        x = x_ref[...]
        t = x[:, :, None] * w_ref[...][None] + b_ref[...][None]
        mu = jnp.mean(t, axis=-1, keepdims=True)
        var = jnp.mean((t - mu) * (t - mu), axis=-1, keepdims=True)
        t = (t - mu) / jnp.sqrt(var + 1e-5)
        t = t * g_ref[...][None] + be_ref[...][None]
        o_ref[...] = t.reshape(x.shape[0] * N_NUM, D)

    Bb = 512
    g2 = ln_gamma.reshape(1, D)
    b2 = ln_beta.reshape(1, D)
    return pl.pallas_call(
        body,
        grid=(BATCH // Bb,),
        in_specs=[
            pl.BlockSpec((Bb, N_NUM), lambda i: (i, 0)),
            pl.BlockSpec((N_NUM, D), lambda i: (0, 0)),
            pl.BlockSpec((N_NUM, D), lambda i: (0, 0)),
            pl.BlockSpec((1, D), lambda i: (0, 0)),
            pl.BlockSpec((1, D), lambda i: (0, 0)),
        ],
        out_specs=pl.BlockSpec((Bb * N_NUM, D), lambda i: (i, 0)),
        out_shape=jax.ShapeDtypeStruct((BATCH * N_NUM, D), jnp.float32),
    )(x_num, num_weight, num_bias, g2, b2)


def _sc_tokens(tab3, xct, numtok):
    mesh = plsc.VectorSubcoreMesh(core_axis_name="c", subcore_axis_name="s")

    @functools.partial(
        pl.kernel,
        mesh=mesh,
        out_type=jax.ShapeDtypeStruct((BATCH * N_TOK, D), jnp.float32),
        scratch_types=[
            pltpu.VMEM((N_CAT, CHUNK), jnp.int32),
            pltpu.VMEM((N_CAT, CHUNK), jnp.int32),
            pltpu.VMEM((NUM_CHUNKS, CHUNK), jnp.int32),
            pltpu.VMEM((CHUNK, D), jnp.float32),
            pltpu.SemaphoreType.DMA,
        ],
        compiler_params=pltpu.CompilerParams(use_tc_tiling_on_sc=False),
    )
    def k(tab_hbm, xct_hbm, num_hbm, out_hbm, gidx, orow, onum, rows, sem):
        cid = lax.axis_index("c")
        sid = lax.axis_index("s")
        wid = sid * NC + cid
        b0 = wid * B_PER_W

        def stage_body(f, carry):
            pltpu.sync_copy(xct_hbm.at[f, pl.ds(b0, CHUNK)], gidx.at[f])
            return carry

        lax.fori_loop(0, N_CAT, stage_body, 0)

        def cat_idx_body(i, carry):
            f = i // (CHUNK // L)
            col = (i % (CHUNK // L)) * L
            ids = gidx[f, pl.ds(col, L)]
            gidx[f, pl.ds(col, L)] = jnp.minimum(jnp.maximum(ids, 0), CARD)
            k_vec = col + lax.iota(jnp.int32, L)
            orow[f, pl.ds(col, L)] = (b0 + k_vec) * N_TOK + N_NUM + f
            return carry

        lax.fori_loop(0, N_CAT * (CHUNK // L), cat_idx_body, 0)

        def num_idx_body(i, carry):
            c = i // (CHUNK // L)
            col = (i % (CHUNK // L)) * L
            j = i * L + lax.iota(jnp.int32, L)
            q = lax.div(j, N_NUM)
            f = j - q * N_NUM
            onum[c, pl.ds(col, L)] = (b0 + q) * N_TOK + f
            return carry

        lax.fori_loop(0, NUM_ROWS_W // L, num_idx_body, 0)

        def cat_dma_body(f, carry):
            pltpu.async_copy(tab_hbm.at[f].at[gidx.at[f]], rows, sem).wait()
            pltpu.async_copy(rows, out_hbm.at[orow.at[f]], sem).wait()
            return carry

        lax.fori_loop(0, N_CAT, cat_dma_body, 0)

        def num_dma_body(c, carry):
            pltpu.sync_copy(num_hbm.at[pl.ds(b0 * N_NUM + c * CHUNK, CHUNK)], rows)
            pltpu.async_copy(rows, out_hbm.at[onum.at[c]], sem).wait()
            return carry

        lax.fori_loop(0, NUM_CHUNKS, num_dma_body, 0)

    return k(tab3, xct, numtok)


def kernel(x_num, x_cat, num_weight, num_bias, ln_gamma, ln_beta, cat_tables):
    numtok = _num_tokens_tc(x_num, num_weight, num_bias, ln_gamma, ln_beta)
    xct = jnp.transpose(x_cat)
    out = _sc_tokens(cat_tables, xct, numtok)
    return out.reshape(BATCH, N_TOK, D)


# speedup vs baseline: 6.5836x; 6.5836x over previous
import functools

import jax
import jax.numpy as jnp
from jax import lax
from jax.experimental import pallas as pl
from jax.experimental.pallas import tpu as pltpu
from jax.experimental.pallas import tpu_sc as plsc

N_NUM = 13
N_CAT = 26
CARD = 100000
D = 64
BATCH = 4096
N_TOK = N_NUM + N_CAT

NC = 2
NS = 16
NW = NC * NS
L = 16

SUB = 512
NSUB = 197
TAIL1 = 195 * SUB
TAIL2 = TAIL1 + 128
UNITS = N_CAT * NSUB
UNITS_LO = UNITS // NW
UNITS_EXTRA = UNITS % NW
PAIR_CAP = BATCH + L
DUMP = 2 * PAIR_CAP
ROWS = 128
IDMASK = (1 << 17) - 1


def _num_tokens_tc(x_num, num_weight, num_bias, ln_gamma, ln_beta):
    def body(x_ref, w_ref, b_ref, g_ref, be_ref, o_ref):
        x = x_ref[...]
        t = x[:, :, None] * w_ref[...][None] + b_ref[...][None]
        mu = jnp.mean(t, axis=-1, keepdims=True)
        var = jnp.mean((t - mu) * (t - mu), axis=-1, keepdims=True)
        t = (t - mu) / jnp.sqrt(var + 1e-5)
        o_ref[...] = t * g_ref[...][None] + be_ref[...][None]

    Bb = 512
    g2 = ln_gamma.reshape(1, D)
    b2 = ln_beta.reshape(1, D)
    return pl.pallas_call(
        body,
        grid=(BATCH // Bb,),
        in_specs=[
            pl.BlockSpec((Bb, N_NUM), lambda i: (i, 0)),
            pl.BlockSpec((N_NUM, D), lambda i: (0, 0)),
            pl.BlockSpec((N_NUM, D), lambda i: (0, 0)),
            pl.BlockSpec((1, D), lambda i: (0, 0)),
            pl.BlockSpec((1, D), lambda i: (0, 0)),
        ],
        out_specs=pl.BlockSpec((Bb, N_NUM, D), lambda i: (i, 0, 0)),
        out_shape=jax.ShapeDtypeStruct((BATCH, N_NUM, D), jnp.float32),
    )(x_num, num_weight, num_bias, g2, b2)


def _scalar(v16):
    return lax.reduce_max(v16, axes=(0,))


def _sc_cat_tokens(tabt, xc3, tail2):
    mesh = plsc.VectorSubcoreMesh(core_axis_name="c", subcore_axis_name="s")

    @functools.partial(
        pl.kernel,
        mesh=mesh,
        out_type=jax.ShapeDtypeStruct((BATCH * N_CAT, 128), jnp.float32),
        scratch_types=[
            pltpu.VMEM((8, BATCH // 8), jnp.int32),
            pltpu.VMEM((2 * PAIR_CAP + L,), jnp.int32),
            pltpu.VMEM((PAIR_CAP + L,), jnp.int32),
            pltpu.VMEM((D, SUB), jnp.float32),
            pltpu.VMEM((ROWS, 128), jnp.float32),
            pltpu.VMEM((1, ROWS), jnp.int32),
            pltpu.SMEM((8,), jnp.int32),
            pltpu.SemaphoreType.DMA,
            pltpu.SemaphoreType.DMA,
        ],
        compiler_params=pltpu.CompilerParams(needs_layout_passes=False),
    )
    def k(tab_hbm, xc_hbm, tail_hbm, out_hbm, idsv, pairs, ulist, seg, rows, orow,
          cnts, sem, sem2):
        cid = lax.axis_index("c")
        sid = lax.axis_index("s")
        wid = sid * NC + cid
        u0 = wid * UNITS_LO + jnp.minimum(wid, UNITS_EXTRA)
        u1 = u0 + UNITS_LO + jnp.where(wid < UNITS_EXTRA, 1, 0)
        f0 = lax.div(u0, jnp.int32(NSUB))
        f1 = lax.div(u1 - 1, jnp.int32(NSUB))
        lanes = lax.iota(jnp.int32, L)
        zeros16 = jnp.zeros((L,), jnp.int32)

        def plane_scan(f, carry):
            fi = f - f0
            slo = jnp.maximum(u0 - f * NSUB, 0)
            shi = jnp.minimum(u1 - f * NSUB, NSUB)
            pltpu.sync_copy(xc_hbm.at[f], idsv)

            def chunk(i, off):
                r = lax.div(i, jnp.int32(32))
                c = (i - r * 32) * L
                ids = idsv[r, pl.ds(c, L)]
                ids = jnp.minimum(jnp.maximum(ids, 0), CARD)
                w = lax.shift_right_logical(ids, 9) + jnp.where(
                    ids >= TAIL2, 1, 0)
                m = (w >= slo) & (w < shi)
                b = i * L + lanes
                packed = ids | (b << 17)
                mi = m.astype(jnp.int32)
                pos = plsc.cumsum(mi) - 1
                dst = jnp.where(m, fi * PAIR_CAP + off + pos, DUMP)
                plsc.store_scatter(pairs, [dst], packed)
                return off + lax.reduce_sum(mi, axes=(0,))

            n = lax.fori_loop(0, BATCH // L, chunk, jnp.int32(0))
            cnts[fi] = n
            return carry

        lax.fori_loop(f0, f1 + 1, plane_scan, 0)

        def unit_body(u, rpos):
            f = lax.div(u, jnp.int32(NSUB))
            s = u - f * NSUB
            fi = f - f0

            @pl.when(s < NSUB - 2)
            def _():
                cps = [
                    pltpu.async_copy(
                        tab_hbm.at[f, pl.ds(bd * 8, 8), pl.ds(s * SUB, SUB)],
                        seg.at[pl.ds(bd * 8, 8), :], sem)
                    for bd in range(8)
                ]
                for cp in cps:
                    cp.wait()

            @pl.when(s == NSUB - 2)
            def _():
                cps = [
                    pltpu.async_copy(
                        tab_hbm.at[f, pl.ds(bd * 8, 8), pl.ds(TAIL1, 128)],
                        seg.at[pl.ds(bd * 8, 8), pl.ds(0, 128)], sem)
                    for bd in range(8)
                ]
                for cp in cps:
                    cp.wait()

            @pl.when(s == NSUB - 1)
            def _():
                pltpu.async_copy(tail_hbm.at[f], seg.at[:, pl.ds(0, 128)],
                                 sem).wait()

            n_f = cnts[fi]

            def cchunk(i, uoff):
                p = pairs[pl.ds(fi * PAIR_CAP + i * L, L)]
                pid = p & jnp.int32(IDMASK)
                valid = (i * L + lanes) < n_f
                w = lax.shift_right_logical(pid, 9) + jnp.where(
                    pid >= TAIL2, 1, 0)
                m = valid & (w == s)
                mi = m.astype(jnp.int32)
                pos = plsc.cumsum(mi) - 1
                dst = jnp.where(m, uoff + pos, PAIR_CAP)
                plsc.store_scatter(ulist, [dst], p)
                return uoff + lax.reduce_sum(mi, axes=(0,))

            n_u = lax.fori_loop(0, lax.div(n_f + (L - 1), jnp.int32(L)),
                                cchunk, jnp.int32(0))

            base = jnp.where(s == NSUB - 1, TAIL2, s * SUB)

            def token(t, rpos2):
                p = plsc.load_gather(ulist, [jnp.full((L,), t, jnp.int32)])
                col = (p & jnp.int32(IDMASK)) - base
                orow_v = lax.shift_right_logical(p, 17) * N_CAT + f
                rr = rpos2 & (ROWS - 1)
                for q in range(D // L):
                    vals = plsc.load_gather(seg, [lanes + q * L, col])
                    rows[rr, pl.ds(q * L, L)] = vals
                plsc.store_scatter(orow, [zeros16, jnp.full((L,), rr, jnp.int32)],
                                   orow_v)

                @pl.when(rr == ROWS - 1)
                def _():
                    pltpu.async_copy(rows, out_hbm.at[orow.at[0]], sem2).wait()

                return rpos2 + 1

            return lax.fori_loop(0, n_u, token, rpos)

        rpos = lax.fori_loop(u0, u1, unit_body, jnp.int32(0))

        rem = rpos & (ROWS - 1)

        @pl.when(rem > 0)
        def _():
            last_or = plsc.load_gather(
                orow, [zeros16, jnp.full((L,), rem - 1, jnp.int32)])

            def padrow(i, c):
                r = rem + i
                for q in range(D // L):
                    rows[r, pl.ds(q * L, L)] = rows[rem - 1, pl.ds(q * L, L)]
                return c

            lax.fori_loop(0, ROWS - rem, padrow, 0)

            def padidx(i, c):
                colv = i * L + lanes
                cur = orow[0, pl.ds(i * L, L)]
                orow[0, pl.ds(i * L, L)] = jnp.where(colv < rem, cur, last_or)
                return c

            lax.fori_loop(0, ROWS // L, padidx, 0)
            pltpu.async_copy(rows, out_hbm.at[orow.at[0]], sem2).wait()

    return k(tabt, xc3, tail2)


def kernel(x_num, x_cat, num_weight, num_bias, ln_gamma, ln_beta, cat_tables):
    numtok = _num_tokens_tc(x_num, num_weight, num_bias, ln_gamma, ln_beta)
    tabt = jnp.transpose(cat_tables, (0, 2, 1))
    xc3 = jnp.transpose(x_cat).reshape(N_CAT, 8, BATCH // 8)
    tail2 = jnp.pad(tabt[:, :, TAIL2:], ((0, 0), (0, 0), (0, 128 - (CARD + 1 - TAIL2))))
    rows = _sc_cat_tokens(tabt, xc3, tail2)
    cat = rows[:, :D].reshape(BATCH, N_CAT, D)
    return jnp.concatenate([numtok, cat], axis=1)


# speedup vs baseline: 14.3125x; 2.1740x over previous
import functools

import jax
import jax.numpy as jnp
from jax import lax
from jax.experimental import pallas as pl
from jax.experimental.pallas import tpu as pltpu
from jax.experimental.pallas import tpu_sc as plsc

N_NUM = 13
N_CAT = 26
CARD = 100000
D = 64
BATCH = 4096
N_TOK = N_NUM + N_CAT

NC = 2
NS = 16
NW = NC * NS
L = 16

SUB = 512
NSUB = 197
TAIL1 = 195 * SUB
TAIL2 = TAIL1 + 128
UNITS = N_CAT * NSUB
UNITS_LO = UNITS // NW
UNITS_EXTRA = UNITS % NW
PAIR_CAP = BATCH + L
DUMP = 2 * PAIR_CAP
ROWS = 128
NCB = 13
IDMASK = (1 << 17) - 1


def _num_tokens_tc(x_num, num_weight, num_bias, ln_gamma, ln_beta):
    def body(x_ref, w_ref, b_ref, g_ref, be_ref, o_ref):
        x = x_ref[...]
        t = x[:, :, None] * w_ref[...][None] + b_ref[...][None]
        mu = jnp.mean(t, axis=-1, keepdims=True)
        var = jnp.mean((t - mu) * (t - mu), axis=-1, keepdims=True)
        t = (t - mu) / jnp.sqrt(var + 1e-5)
        o_ref[...] = t * g_ref[...][None] + be_ref[...][None]

    Bb = 512
    g2 = ln_gamma.reshape(1, D)
    b2 = ln_beta.reshape(1, D)
    return pl.pallas_call(
        body,
        grid=(BATCH // Bb,),
        in_specs=[
            pl.BlockSpec((Bb, N_NUM), lambda i: (i, 0)),
            pl.BlockSpec((N_NUM, D), lambda i: (0, 0)),
            pl.BlockSpec((N_NUM, D), lambda i: (0, 0)),
            pl.BlockSpec((1, D), lambda i: (0, 0)),
            pl.BlockSpec((1, D), lambda i: (0, 0)),
        ],
        out_specs=pl.BlockSpec((Bb, N_NUM, D), lambda i: (i, 0, 0)),
        out_shape=jax.ShapeDtypeStruct((BATCH, N_NUM, D), jnp.float32),
    )(x_num, num_weight, num_bias, g2, b2)


def _scalar(v16):
    return lax.reduce_max(v16, axes=(0,))


def _sc_cat_tokens(tabt, xc3, tail2):
    mesh = plsc.VectorSubcoreMesh(core_axis_name="c", subcore_axis_name="s")

    @functools.partial(
        pl.kernel,
        mesh=mesh,
        out_type=jax.ShapeDtypeStruct((BATCH * N_CAT, 128), jnp.float32),
        scratch_types=[
            pltpu.VMEM((8, BATCH // 8), jnp.int32),
            pltpu.VMEM((2 * PAIR_CAP + L,), jnp.int32),
            pltpu.VMEM((2 * PAIR_CAP + L,), jnp.int32),
            pltpu.VMEM((PAIR_CAP + L,), jnp.int32),
            pltpu.VMEM((D, SUB), jnp.float32),
            pltpu.VMEM((D, SUB), jnp.float32),
            pltpu.VMEM((ROWS, 128), jnp.float32),
            pltpu.VMEM((1, ROWS), jnp.int32),
            pltpu.SMEM((64,), jnp.int32),
            pltpu.SemaphoreType.DMA,
            pltpu.SemaphoreType.DMA,
            pltpu.SemaphoreType.DMA,
        ],
        compiler_params=pltpu.CompilerParams(needs_layout_passes=False),
    )
    def k(tab_hbm, xc_hbm, tail_hbm, out_hbm, idsv, pairs, coarse, ulist,
          sega, segb, rows, orow, cbs, sema, semb, sem2):
        cid = lax.axis_index("c")
        sid = lax.axis_index("s")
        wid = sid * NC + cid
        u0 = wid * UNITS_LO + jnp.minimum(wid, UNITS_EXTRA)
        u1 = u0 + UNITS_LO + jnp.where(wid < UNITS_EXTRA, 1, 0)
        f0 = lax.div(u0, jnp.int32(NSUB))
        f1 = lax.div(u1 - 1, jnp.int32(NSUB))
        lanes = lax.iota(jnp.int32, L)
        zeros16 = jnp.zeros((L,), jnp.int32)

        def stage(u, seg, sem):
            f = lax.div(u, jnp.int32(NSUB))
            s = u - f * NSUB

            @pl.when(s < NSUB - 2)
            def _():
                pltpu.async_copy(tab_hbm.at[f, :, pl.ds(s * SUB, SUB)], seg,
                                 sem)

            @pl.when(s == NSUB - 2)
            def _():
                pltpu.async_copy(tab_hbm.at[f, :, pl.ds(TAIL1, 128)],
                                 seg.at[:, pl.ds(0, 128)], sem)

            @pl.when(s == NSUB - 1)
            def _():
                pltpu.async_copy(tail_hbm.at[f], seg.at[:, pl.ds(0, 128)], sem)

        def wait_stage(u, seg, sem):
            f = lax.div(u, jnp.int32(NSUB))
            s = u - f * NSUB

            @pl.when(s < NSUB - 2)
            def _():
                pltpu.make_async_copy(tab_hbm.at[f, :, pl.ds(s * SUB, SUB)],
                                      seg, sem).wait()

            @pl.when(s >= NSUB - 2)
            def _():
                pltpu.make_async_copy(tail_hbm.at[f],
                                      seg.at[:, pl.ds(0, 128)], sem).wait()

        stage(u0, sega, sema)

        def plane_scan(f, carry):
            fi = f - f0
            slo = jnp.maximum(u0 - f * NSUB, 0)
            shi = jnp.minimum(u1 - f * NSUB, NSUB)
            pltpu.sync_copy(xc_hbm.at[f], idsv)

            def chunk(i, off):
                r = lax.div(i, jnp.int32(32))
                c = (i - r * 32) * L
                ids = idsv[r, pl.ds(c, L)]
                ids = jnp.minimum(jnp.maximum(ids, 0), CARD)
                w = lax.shift_right_logical(ids, 9) + jnp.where(
                    ids >= TAIL2, 1, 0)
                m = (w >= slo) & (w < shi)
                b = i * L + lanes
                packed = ids | (b << 17)
                mi = m.astype(jnp.int32)
                pos = plsc.cumsum(mi) - 1
                dst = jnp.where(m, fi * PAIR_CAP + off + pos, DUMP)
                plsc.store_scatter(pairs, [dst], packed)
                return off + lax.reduce_sum(mi, axes=(0,))

            n = lax.fori_loop(0, BATCH // L, chunk, jnp.int32(0))
            cbs[fi * 16 + 14] = n
            return carry

        lax.fori_loop(f0, f1 + 1, plane_scan, 0)

        def plane_bucket(f, carry):
            fi = f - f0
            n_f = cbs[fi * 16 + 14]
            ntrip = lax.div(n_f + (L - 1), jnp.int32(L))

            def cbloop(cb, off):
                cbs[fi * 16 + cb] = off

                def ch(i, o2):
                    p = pairs[pl.ds(fi * PAIR_CAP + i * L, L)]
                    pid = p & jnp.int32(IDMASK)
                    valid = (i * L + lanes) < n_f
                    w = lax.shift_right_logical(pid, 9) + jnp.where(
                        pid >= TAIL2, 1, 0)
                    m = valid & (lax.shift_right_logical(w, 4) == cb)
                    mi = m.astype(jnp.int32)
                    pos = plsc.cumsum(mi) - 1
                    dst = jnp.where(m, o2 + pos, DUMP)
                    plsc.store_scatter(coarse, [dst], p)
                    return o2 + lax.reduce_sum(mi, axes=(0,))

                return lax.fori_loop(0, ntrip, ch, off)

            offf = lax.fori_loop(0, NCB, cbloop, fi * PAIR_CAP)
            cbs[fi * 16 + NCB] = offf
            return carry

        lax.fori_loop(f0, f1 + 1, plane_bucket, 0)

        def process(u, seg, rpos):
            valid_u = u < u1
            f = lax.div(u, jnp.int32(NSUB))
            s = u - f * NSUB
            fi = jnp.minimum(f - f0, 1)
            cb = lax.shift_right_logical(s, 4)
            cstart = cbs[fi * 16 + cb]
            cend = cbs[fi * 16 + cb + 1]
            cn = jnp.where(valid_u, cend - cstart, 0)

            def cchunk(i, uoff):
                p = coarse[pl.ds(cstart + i * L, L)]
                pid = p & jnp.int32(IDMASK)
                valid = (i * L + lanes) < cn
                w = lax.shift_right_logical(pid, 9) + jnp.where(
                    pid >= TAIL2, 1, 0)
                m = valid & (w == s)
                mi = m.astype(jnp.int32)
                pos = plsc.cumsum(mi) - 1
                dst = jnp.where(m, uoff + pos, PAIR_CAP)
                plsc.store_scatter(ulist, [dst], p)
                return uoff + lax.reduce_sum(mi, axes=(0,))

            n_u = lax.fori_loop(0, lax.div(cn + (L - 1), jnp.int32(L)),
                                cchunk, jnp.int32(0))
            base = jnp.where(s == NSUB - 1, TAIL2, s * SUB)

            def token(t, rpos2):
                p = plsc.load_gather(ulist, [jnp.full((L,), t, jnp.int32)])
                col = (p & jnp.int32(IDMASK)) - base
                orow_v = lax.shift_right_logical(p, 17) * N_CAT + f
                rr = rpos2 & (ROWS - 1)
                for q in range(D // L):
                    vals = plsc.load_gather(seg, [lanes + q * L, col])
                    rows[rr, pl.ds(q * L, L)] = vals
                plsc.store_scatter(
                    orow, [zeros16, jnp.full((L,), rr, jnp.int32)], orow_v)

                @pl.when(rr == ROWS - 1)
                def _():
                    pltpu.async_copy(rows, out_hbm.at[orow.at[0]], sem2).wait()

                return rpos2 + 1

            return lax.fori_loop(0, n_u, token, rpos)

        def pair_body(kk, rpos):
            ua = u0 + 2 * kk
            ub = ua + 1

            wait_stage(ua, sega, sema)

            @pl.when(ub < u1)
            def _():
                stage(ub, segb, semb)

            rpos = process(ua, sega, rpos)

            @pl.when(ub < u1)
            def _():
                wait_stage(ub, segb, semb)

            @pl.when(ub + 1 < u1)
            def _():
                stage(ub + 1, sega, sema)

            return process(ub, segb, rpos)

        npair = lax.div(u1 - u0 + 1, jnp.int32(2))
        rpos = lax.fori_loop(0, npair, pair_body, jnp.int32(0))

        rem = rpos & (ROWS - 1)

        @pl.when(rem > 0)
        def _():
            last_or = plsc.load_gather(
                orow, [zeros16, jnp.full((L,), rem - 1, jnp.int32)])

            def padrow(i, c):
                r = rem + i
                for q in range(D // L):
                    rows[r, pl.ds(q * L, L)] = rows[rem - 1, pl.ds(q * L, L)]
                return c

            lax.fori_loop(0, ROWS - rem, padrow, 0)

            def padidx(i, c):
                colv = i * L + lanes
                cur = orow[0, pl.ds(i * L, L)]
                orow[0, pl.ds(i * L, L)] = jnp.where(colv < rem, cur, last_or)
                return c

            lax.fori_loop(0, ROWS // L, padidx, 0)
            pltpu.async_copy(rows, out_hbm.at[orow.at[0]], sem2).wait()

    return k(tabt, xc3, tail2)


def kernel(x_num, x_cat, num_weight, num_bias, ln_gamma, ln_beta, cat_tables):
    numtok = _num_tokens_tc(x_num, num_weight, num_bias, ln_gamma, ln_beta)
    tabt = jnp.transpose(cat_tables, (0, 2, 1))
    xc3 = jnp.transpose(x_cat).reshape(N_CAT, 8, BATCH // 8)
    tail2 = jnp.pad(tabt[:, :, TAIL2:], ((0, 0), (0, 0), (0, 128 - (CARD + 1 - TAIL2))))
    rows = _sc_cat_tokens(tabt, xc3, tail2)
    cat = rows[:, :D].reshape(BATCH, N_CAT, D)
    return jnp.concatenate([numtok, cat], axis=1)
